# Initial kernel scaffold; baseline (speedup 1.0000x reference)
import jax
import jax.numpy as jnp
from jax.experimental import pallas as pl


def kernel(x, edge_index, edge_attr, W_rel, b_rel, W_root, W_fc, b_fc):
    raise NotImplementedError("write your pallas kernel here")


# speedup vs baseline: 4.3074x; 4.3074x over previous
import functools

import jax
import jax.numpy as jnp
from jax import lax
from jax.experimental import pallas as pl
from jax.experimental.pallas import tpu as pltpu
from jax.experimental.pallas import tpu_sc as plsc

N_NODES = 10000
N_PAD = 10240
D_HALF = 128
N_EDGES = 160000
EDGES_PER_TILE = 10000
CHUNK = 80
N_CHUNKS = EDGES_PER_TILE // CHUNK
STRIPE = N_PAD // 16


def _sc_body(xs_hbm, src_hbm, dst_hbm, attr_hbm, out_hbm,
             src_all, dst_all, attr_all, gidx_v, dst_v, rows_v,
             agg_sh, sem):
    c = lax.axis_index("c")
    s = lax.axis_index("s")
    stripe_base = s * STRIPE

    def zero_rows(i, carry):
        for f in range(8):
            rows_v[i, pl.ds(16 * f, 16)] = jnp.zeros((16,), jnp.float32)
        return carry

    lax.fori_loop(0, CHUNK, zero_rows, 0)

    def zero_stripe(r, carry):
        pltpu.sync_copy(rows_v, agg_sh.at[pl.ds(stripe_base + r * CHUNK, CHUNK)])
        return carry

    lax.fori_loop(0, STRIPE // CHUNK, zero_stripe, 0)
    plsc.subcore_barrier()

    ebase = s * EDGES_PER_TILE
    pltpu.sync_copy(src_hbm.at[pl.ds(ebase, EDGES_PER_TILE)], src_all)
    pltpu.sync_copy(dst_hbm.at[pl.ds(ebase, EDGES_PER_TILE)], dst_all)
    pltpu.sync_copy(attr_hbm.at[pl.ds(ebase, EDGES_PER_TILE)], attr_all)

    coff = c * N_NODES

    def do_chunk(j, carry):
        o = j * CHUNK
        for v in range(CHUNK // 16):
            sl = pl.ds(o + 16 * v, 16)
            w = pl.ds(16 * v, 16)
            gidx_v[w] = src_all[sl] + coff
            dst_v[w] = dst_all[sl]
        pltpu.async_copy(xs_hbm.at[gidx_v], rows_v, sem).wait()

        def scale(g, carry2):
            avec = attr_all[pl.ds(o + g * 16, 16)]
            for l in range(16):
                a = avec[l]
                row = g * 16 + l
                for f in range(8):
                    rows_v[row, pl.ds(16 * f, 16)] = (
                        rows_v[row, pl.ds(16 * f, 16)] * a
                    )
            return carry2

        lax.fori_loop(0, CHUNK // 16, scale, 0)

        pltpu.sync_copy(rows_v, agg_sh.at[dst_v], add=True)
        return carry

    lax.fori_loop(0, N_CHUNKS, do_chunk, 0)
    plsc.subcore_barrier()

    def writeback(r, carry):
        b = stripe_base + r * CHUNK
        pltpu.sync_copy(agg_sh.at[pl.ds(b, CHUNK)], out_hbm.at[c, pl.ds(b, CHUNK)])
        return carry

    lax.fori_loop(0, STRIPE // CHUNK, writeback, 0)


@jax.jit
def _sc_segment(xs, src, dst, attr):
    mesh = plsc.VectorSubcoreMesh(core_axis_name="c", subcore_axis_name="s",
                                  num_cores=2, num_subcores=16)
    f = pl.kernel(
        _sc_body,
        out_type=jax.ShapeDtypeStruct((2, N_PAD, D_HALF), jnp.float32),
        mesh=mesh,
        scratch_types=[
            pltpu.VMEM((EDGES_PER_TILE,), jnp.int32),
            pltpu.VMEM((EDGES_PER_TILE,), jnp.int32),
            pltpu.VMEM((EDGES_PER_TILE,), jnp.float32),
            pltpu.VMEM((CHUNK,), jnp.int32),
            pltpu.VMEM((CHUNK,), jnp.int32),
            pltpu.VMEM((CHUNK, D_HALF), jnp.float32),
            pltpu.VMEM_SHARED((N_PAD, D_HALF), jnp.float32),
            pltpu.SemaphoreType.DMA,
        ],
        name="gnn_segment_sum_sc",
    )
    return f(xs, src, dst, attr)


def _tc_body(aggh_ref, x_ref, wrel_ref, wroot_ref, wfc_ref, brel_ref,
             bfc_ref, out_ref):
    a = aggh_ref[...]
    h = jnp.dot(a[0], wrel_ref[0], preferred_element_type=jnp.float32)
    h += jnp.dot(a[1], wrel_ref[1], preferred_element_type=jnp.float32)
    h += jnp.dot(x_ref[...], wroot_ref[...], preferred_element_type=jnp.float32)
    h += brel_ref[...]
    h = jnp.maximum(h, 0.0)
    out_ref[...] = (
        jnp.dot(h, wfc_ref[...], preferred_element_type=jnp.float32)
        + bfc_ref[...]
    )


@functools.partial(jax.jit, static_argnames=())
def _tc_dense(aggh, x, wrelT3, wrootT, wfcT, brel, bfc):
    n, d_in = x.shape
    d_hid = wrootT.shape[1]
    n_cls = wfcT.shape[1]
    blk = 1000
    grid = (n // blk,)
    return pl.pallas_call(
        _tc_body,
        grid=grid,
        in_specs=[
            pl.BlockSpec((2, blk, D_HALF), lambda i: (0, i, 0)),
            pl.BlockSpec((blk, d_in), lambda i: (i, 0)),
            pl.BlockSpec((2, D_HALF, d_hid), lambda i: (0, 0, 0)),
            pl.BlockSpec((d_in, d_hid), lambda i: (0, 0)),
            pl.BlockSpec((d_hid, n_cls), lambda i: (0, 0)),
            pl.BlockSpec((1, d_hid), lambda i: (0, 0)),
            pl.BlockSpec((1, n_cls), lambda i: (0, 0)),
        ],
        out_specs=pl.BlockSpec((blk, n_cls), lambda i: (i, 0)),
        out_shape=jax.ShapeDtypeStruct((n, n_cls), jnp.float32),
    )(aggh, x, wrelT3, wrootT, wfcT, brel, bfc)


def kernel(x, edge_index, edge_attr, W_rel, b_rel, W_root, W_fc, b_fc):
    src = edge_index[0]
    dst = edge_index[1]
    xs = jnp.concatenate([x[:, :D_HALF], x[:, D_HALF:]], axis=0)
    aggh = _sc_segment(xs, src, dst, edge_attr)
    wrelT3 = W_rel.T.reshape(2, D_HALF, -1)
    out = _tc_dense(aggh, x, wrelT3, W_root.T, W_fc.T,
                    b_rel[None, :], b_fc[None, :])
    return out


# speedup vs baseline: 4.3695x; 1.0144x over previous
import functools

import jax
import jax.numpy as jnp
from jax import lax
from jax.experimental import pallas as pl
from jax.experimental.pallas import tpu as pltpu
from jax.experimental.pallas import tpu_sc as plsc

N_NODES = 10000
N_PAD = 10240
D_HALF = 128
N_EDGES = 160000
EDGES_PER_TILE = 10000
CHUNK = 80
N_CHUNKS = EDGES_PER_TILE // CHUNK
STRIPE = N_PAD // 16


NBUF = 4


def _sc_body(xs_hbm, src_hbm, dst_hbm, attr_hbm, out_hbm,
             sbuf0, sbuf1, sbuf2, sbuf3,
             gidx0, gidx1, gidx2, gidx3,
             dstb0, dstb1, dstb2, dstb3,
             abuf0, abuf1, abuf2, abuf3,
             rows0, rows1, rows2, rows3,
             agg_sh,
             gsem0, gsem1, gsem2, gsem3,
             ssem0, ssem1, ssem2, ssem3,
             isem0, isem1, isem2, isem3):
    sbuf = [sbuf0, sbuf1, sbuf2, sbuf3]
    gidx = [gidx0, gidx1, gidx2, gidx3]
    dstb = [dstb0, dstb1, dstb2, dstb3]
    abuf = [abuf0, abuf1, abuf2, abuf3]
    rows = [rows0, rows1, rows2, rows3]
    gsem = [gsem0, gsem1, gsem2, gsem3]
    ssem = [ssem0, ssem1, ssem2, ssem3]
    isem = [isem0, isem1, isem2, isem3]

    c = lax.axis_index("c")
    s = lax.axis_index("s")
    stripe_base = s * STRIPE
    ebase = s * EDGES_PER_TILE
    coff = c * N_NODES

    def fire_idx(b, cidx):
        o = ebase + cidx * CHUNK
        pltpu.async_copy(src_hbm.at[pl.ds(o, CHUNK)], sbuf[b], isem[b])
        pltpu.async_copy(dst_hbm.at[pl.ds(o, CHUNK)], dstb[b], isem[b])
        pltpu.async_copy(attr_hbm.at[pl.ds(o, CHUNK)], abuf[b], isem[b])

    def wait_idx(b, cidx):
        o = ebase + cidx * CHUNK
        pltpu.make_async_copy(src_hbm.at[pl.ds(o, CHUNK)], sbuf[b],
                              isem[b]).wait()
        pltpu.make_async_copy(dst_hbm.at[pl.ds(o, CHUNK)], dstb[b],
                              isem[b]).wait()
        pltpu.make_async_copy(attr_hbm.at[pl.ds(o, CHUNK)], abuf[b],
                              isem[b]).wait()

    def build_gidx(b):
        for v in range(CHUNK // 16):
            gidx[b][pl.ds(16 * v, 16)] = sbuf[b][pl.ds(16 * v, 16)] + coff

    def scale_buf(b):
        rb = rows[b]
        ab = abuf[b]

        def scale(g, carry2):
            avec = ab[pl.ds(g * 16, 16)]
            for l in range(16):
                a = avec[l]
                row = g * 16 + l
                for f in range(8):
                    rb[row, pl.ds(16 * f, 16)] = rb[row, pl.ds(16 * f, 16)] * a
            return carry2

        lax.fori_loop(0, CHUNK // 16, scale, 0)

    def zero_rows(i, carry):
        for f in range(8):
            rows0[i, pl.ds(16 * f, 16)] = jnp.zeros((16,), jnp.float32)
        return carry

    lax.fori_loop(0, CHUNK, zero_rows, 0)

    def zero_stripe(r, carry):
        pltpu.sync_copy(rows0, agg_sh.at[pl.ds(stripe_base + r * CHUNK, CHUNK)])
        return carry

    lax.fori_loop(0, STRIPE // CHUNK, zero_stripe, 0)
    plsc.subcore_barrier()

    fire_idx(0, 0)
    fire_idx(1, 1)
    wait_idx(0, 0)
    build_gidx(0)
    pltpu.async_copy(xs_hbm.at[gidx[0]], rows[0], gsem[0])

    def slot_group(t, carry):
        for u in range(NBUF):
            j = NBUF * t + u
            b = u
            b1 = (u + 1) % NBUF
            bq = (u + 2) % NBUF

            @pl.when(j < N_CHUNKS)
            def _():
                pltpu.make_async_copy(xs_hbm.at[gidx[b]], rows[b],
                                      gsem[b]).wait()
                scale_buf(b)
                pltpu.sync_copy(rows[b], agg_sh.at[dstb[b]], add=True)

                @pl.when(j + 2 < N_CHUNKS)
                def _():
                    fire_idx(bq, j + 2)

                @pl.when(j + 1 < N_CHUNKS)
                def _():
                    wait_idx(b1, j + 1)
                    build_gidx(b1)
                    pltpu.async_copy(xs_hbm.at[gidx[b1]], rows[b1], gsem[b1])

        return carry

    lax.fori_loop(0, (N_CHUNKS + NBUF - 1) // NBUF, slot_group, 0)
    plsc.subcore_barrier()

    def writeback(r, carry):
        b = stripe_base + r * CHUNK
        pltpu.sync_copy(agg_sh.at[pl.ds(b, CHUNK)], out_hbm.at[c, pl.ds(b, CHUNK)])
        return carry

    lax.fori_loop(0, STRIPE // CHUNK, writeback, 0)


@jax.jit
def _sc_segment(xs, src, dst, attr):
    mesh = plsc.VectorSubcoreMesh(core_axis_name="c", subcore_axis_name="s",
                                  num_cores=2, num_subcores=16)
    f = pl.kernel(
        _sc_body,
        out_type=jax.ShapeDtypeStruct((2, N_PAD, D_HALF), jnp.float32),
        mesh=mesh,
        scratch_types=(
            [pltpu.VMEM((CHUNK,), jnp.int32) for _ in range(3 * NBUF)]
            + [pltpu.VMEM((CHUNK,), jnp.float32) for _ in range(NBUF)]
            + [pltpu.VMEM((CHUNK, D_HALF), jnp.float32) for _ in range(NBUF)]
            + [pltpu.VMEM_SHARED((N_PAD, D_HALF), jnp.float32)]
            + [pltpu.SemaphoreType.DMA for _ in range(3 * NBUF)]
        ),
        name="gnn_segment_sum_sc",
    )
    return f(xs, src, dst, attr)


def _tc_body(aggh_ref, x_ref, wrel_ref, wroot_ref, wfc_ref, brel_ref,
             bfc_ref, out_ref):
    a = aggh_ref[...]
    h = jnp.dot(a[0], wrel_ref[0], preferred_element_type=jnp.float32)
    h += jnp.dot(a[1], wrel_ref[1], preferred_element_type=jnp.float32)
    h += jnp.dot(x_ref[...], wroot_ref[...], preferred_element_type=jnp.float32)
    h += brel_ref[...]
    h = jnp.maximum(h, 0.0)
    out_ref[...] = (
        jnp.dot(h, wfc_ref[...], preferred_element_type=jnp.float32)
        + bfc_ref[...]
    )


@functools.partial(jax.jit, static_argnames=())
def _tc_dense(aggh, x, wrelT3, wrootT, wfcT, brel, bfc):
    n, d_in = x.shape
    d_hid = wrootT.shape[1]
    n_cls = wfcT.shape[1]
    blk = 1000
    grid = (n // blk,)
    return pl.pallas_call(
        _tc_body,
        grid=grid,
        in_specs=[
            pl.BlockSpec((2, blk, D_HALF), lambda i: (0, i, 0)),
            pl.BlockSpec((blk, d_in), lambda i: (i, 0)),
            pl.BlockSpec((2, D_HALF, d_hid), lambda i: (0, 0, 0)),
            pl.BlockSpec((d_in, d_hid), lambda i: (0, 0)),
            pl.BlockSpec((d_hid, n_cls), lambda i: (0, 0)),
            pl.BlockSpec((1, d_hid), lambda i: (0, 0)),
            pl.BlockSpec((1, n_cls), lambda i: (0, 0)),
        ],
        out_specs=pl.BlockSpec((blk, n_cls), lambda i: (i, 0)),
        out_shape=jax.ShapeDtypeStruct((n, n_cls), jnp.float32),
    )(aggh, x, wrelT3, wrootT, wfcT, brel, bfc)


def kernel(x, edge_index, edge_attr, W_rel, b_rel, W_root, W_fc, b_fc):
    src = edge_index[0]
    dst = edge_index[1]
    xs = jnp.concatenate([x[:, :D_HALF], x[:, D_HALF:]], axis=0)
    aggh = _sc_segment(xs, src, dst, edge_attr)
    wrelT3 = W_rel.T.reshape(2, D_HALF, -1)
    out = _tc_dense(aggh, x, wrelT3, W_root.T, W_fc.T,
                    b_rel[None, :], b_fc[None, :])
    return out


# speedup vs baseline: 5.2233x; 1.1954x over previous
import functools

import jax
import jax.numpy as jnp
from jax import lax
from jax.experimental import pallas as pl
from jax.experimental.pallas import tpu as pltpu
from jax.experimental.pallas import tpu_sc as plsc

N_NODES = 10000
N_PAD = 10240
D_HALF = 128
N_EDGES = 160000
EDGES_PER_TILE = 10000
CHUNK = 80
N_CHUNKS = EDGES_PER_TILE // CHUNK
STRIPE = N_PAD // 16


NBUF = 4


def _sc_body(xs_hbm, src_hbm, dst_hbm, attr_hbm, out_hbm,
             sbuf0, sbuf1, sbuf2, sbuf3,
             gidx0, gidx1, gidx2, gidx3,
             dstb0, dstb1, dstb2, dstb3,
             abuf0, abuf1, abuf2, abuf3,
             rows0, rows1, rows2, rows3,
             agg_sh,
             gsem0, gsem1, gsem2, gsem3,
             ssem0, ssem1, ssem2, ssem3,
             isem0, isem1, isem2, isem3):
    sbuf = [sbuf0, sbuf1, sbuf2, sbuf3]
    gidx = [gidx0, gidx1, gidx2, gidx3]
    dstb = [dstb0, dstb1, dstb2, dstb3]
    abuf = [abuf0, abuf1, abuf2, abuf3]
    rows = [rows0, rows1, rows2, rows3]
    gsem = [gsem0, gsem1, gsem2, gsem3]
    ssem = [ssem0, ssem1, ssem2, ssem3]
    isem = [isem0, isem1, isem2, isem3]

    c = lax.axis_index("c")
    s = lax.axis_index("s")
    stripe_base = s * STRIPE
    ebase = s * EDGES_PER_TILE
    coff = c * N_NODES

    def fire_idx(b, cidx):
        o = ebase + cidx * CHUNK
        pltpu.async_copy(src_hbm.at[pl.ds(o, CHUNK)], sbuf[b], isem[b])
        pltpu.async_copy(dst_hbm.at[pl.ds(o, CHUNK)], dstb[b], isem[b])
        pltpu.async_copy(attr_hbm.at[pl.ds(o, CHUNK)], abuf[b], isem[b])

    def wait_idx(b, cidx):
        o = ebase + cidx * CHUNK
        pltpu.make_async_copy(src_hbm.at[pl.ds(o, CHUNK)], sbuf[b],
                              isem[b]).wait()
        pltpu.make_async_copy(dst_hbm.at[pl.ds(o, CHUNK)], dstb[b],
                              isem[b]).wait()
        pltpu.make_async_copy(attr_hbm.at[pl.ds(o, CHUNK)], abuf[b],
                              isem[b]).wait()

    def build_gidx(b):
        for v in range(CHUNK // 16):
            gidx[b][pl.ds(16 * v, 16)] = sbuf[b][pl.ds(16 * v, 16)] + coff

    def scale_buf(b):
        rb = rows[b]
        ab = abuf[b]

        def scale(g, carry2):
            avec = ab[pl.ds(g * 16, 16)]
            for l in range(16):
                a = avec[l]
                row = g * 16 + l
                for f in range(8):
                    rb[row, pl.ds(16 * f, 16)] = rb[row, pl.ds(16 * f, 16)] * a
            return carry2

        lax.fori_loop(0, CHUNK // 16, scale, 0)

    def zero_rows(i, carry):
        for f in range(8):
            rows0[i, pl.ds(16 * f, 16)] = jnp.zeros((16,), jnp.float32)
        return carry

    lax.fori_loop(0, CHUNK, zero_rows, 0)

    def zero_stripe(r, carry):
        pltpu.sync_copy(rows0, agg_sh.at[pl.ds(stripe_base + r * CHUNK, CHUNK)])
        return carry

    lax.fori_loop(0, STRIPE // CHUNK, zero_stripe, 0)
    plsc.subcore_barrier()

    fire_idx(0, 0)
    fire_idx(1, 1)
    wait_idx(0, 0)
    build_gidx(0)
    pltpu.async_copy(xs_hbm.at[gidx[0]], rows[0], gsem[0])

    def slot_group(t, carry):
        for u in range(NBUF):
            j = NBUF * t + u
            b = u
            b1 = (u + 1) % NBUF
            bq = (u + 2) % NBUF

            bp = (u + 3) % NBUF

            @pl.when(j < N_CHUNKS)
            def _():
                pltpu.make_async_copy(xs_hbm.at[gidx[b]], rows[b],
                                      gsem[b]).wait()
                scale_buf(b)

                @pl.when(j >= 1)
                def _():
                    pltpu.make_async_copy(rows[bp], agg_sh.at[dstb[bp]],
                                          ssem[bp]).wait()

                pltpu.async_copy(rows[b], agg_sh.at[dstb[b]], ssem[b],
                                 add=True)

                @pl.when(j + 2 < N_CHUNKS)
                def _():
                    fire_idx(bq, j + 2)

                @pl.when(j + 1 < N_CHUNKS)
                def _():
                    wait_idx(b1, j + 1)
                    build_gidx(b1)
                    pltpu.async_copy(xs_hbm.at[gidx[b1]], rows[b1], gsem[b1])

        return carry

    lax.fori_loop(0, (N_CHUNKS + NBUF - 1) // NBUF, slot_group, 0)

    b_last = (N_CHUNKS - 1) % NBUF
    pltpu.make_async_copy(rows[b_last], agg_sh.at[dstb[b_last]],
                          ssem[b_last]).wait()
    plsc.subcore_barrier()

    def writeback(r, carry):
        b = stripe_base + r * CHUNK
        pltpu.sync_copy(agg_sh.at[pl.ds(b, CHUNK)], out_hbm.at[c, pl.ds(b, CHUNK)])
        return carry

    lax.fori_loop(0, STRIPE // CHUNK, writeback, 0)


@jax.jit
def _sc_segment(xs, src, dst, attr):
    mesh = plsc.VectorSubcoreMesh(core_axis_name="c", subcore_axis_name="s",
                                  num_cores=2, num_subcores=16)
    f = pl.kernel(
        _sc_body,
        out_type=jax.ShapeDtypeStruct((2, N_PAD, D_HALF), jnp.float32),
        mesh=mesh,
        scratch_types=(
            [pltpu.VMEM((CHUNK,), jnp.int32) for _ in range(3 * NBUF)]
            + [pltpu.VMEM((CHUNK,), jnp.float32) for _ in range(NBUF)]
            + [pltpu.VMEM((CHUNK, D_HALF), jnp.float32) for _ in range(NBUF)]
            + [pltpu.VMEM_SHARED((N_PAD, D_HALF), jnp.float32)]
            + [pltpu.SemaphoreType.DMA for _ in range(3 * NBUF)]
        ),
        name="gnn_segment_sum_sc",
    )
    return f(xs, src, dst, attr)


def _tc_body(aggh_ref, x_ref, wrel_ref, wroot_ref, wfc_ref, brel_ref,
             bfc_ref, out_ref):
    a = aggh_ref[...]
    h = jnp.dot(a[0], wrel_ref[0], preferred_element_type=jnp.float32)
    h += jnp.dot(a[1], wrel_ref[1], preferred_element_type=jnp.float32)
    h += jnp.dot(x_ref[...], wroot_ref[...], preferred_element_type=jnp.float32)
    h += brel_ref[...]
    h = jnp.maximum(h, 0.0)
    out_ref[...] = (
        jnp.dot(h, wfc_ref[...], preferred_element_type=jnp.float32)
        + bfc_ref[...]
    )


@functools.partial(jax.jit, static_argnames=())
def _tc_dense(aggh, x, wrelT3, wrootT, wfcT, brel, bfc):
    n, d_in = x.shape
    d_hid = wrootT.shape[1]
    n_cls = wfcT.shape[1]
    blk = 1000
    grid = (n // blk,)
    return pl.pallas_call(
        _tc_body,
        grid=grid,
        in_specs=[
            pl.BlockSpec((2, blk, D_HALF), lambda i: (0, i, 0)),
            pl.BlockSpec((blk, d_in), lambda i: (i, 0)),
            pl.BlockSpec((2, D_HALF, d_hid), lambda i: (0, 0, 0)),
            pl.BlockSpec((d_in, d_hid), lambda i: (0, 0)),
            pl.BlockSpec((d_hid, n_cls), lambda i: (0, 0)),
            pl.BlockSpec((1, d_hid), lambda i: (0, 0)),
            pl.BlockSpec((1, n_cls), lambda i: (0, 0)),
        ],
        out_specs=pl.BlockSpec((blk, n_cls), lambda i: (i, 0)),
        out_shape=jax.ShapeDtypeStruct((n, n_cls), jnp.float32),
    )(aggh, x, wrelT3, wrootT, wfcT, brel, bfc)


def kernel(x, edge_index, edge_attr, W_rel, b_rel, W_root, W_fc, b_fc):
    src = edge_index[0]
    dst = edge_index[1]
    xs = jnp.concatenate([x[:, :D_HALF], x[:, D_HALF:]], axis=0)
    aggh = _sc_segment(xs, src, dst, edge_attr)
    wrelT3 = W_rel.T.reshape(2, D_HALF, -1)
    out = _tc_dense(aggh, x, wrelT3, W_root.T, W_fc.T,
                    b_rel[None, :], b_fc[None, :])
    return out
